# baseline (device time: 234660 ns/iter reference)
import jax
import jax.numpy as jnp
from jax import lax
from jax.experimental import pallas as pl
from jax.experimental.pallas import tpu as pltpu

SIZES = [128, 128, 256, 512] + [1024] * 6 + [512, 256, 128, 128]
OFFS = [sum(SIZES[:i]) for i in range(len(SIZES))]
N = len(SIZES)
MAXC = max(SIZES)
S = 4


def kernel(x):
    m, n = x.shape
    half = m // 2
    assert sum(SIZES) == half

    def body(x_hbm, out_hbm, in_vmem, send_x, recv_x, sum_buf,
             load_sems, store_sems, sx_send, sx_recv, sy_send, sy_recv,
             credit_x, credit_y):
        my_x = lax.axis_index("x")
        my_y = lax.axis_index("y")
        row0 = my_y * half
        x_peer = (1 - my_x, my_y)
        y_peer = (my_x, 1 - my_y)

        def load(c):
            return pltpu.make_async_copy(
                x_hbm.at[pl.ds(row0 + OFFS[c], SIZES[c]), :],
                in_vmem.at[c % 2, pl.ds(0, SIZES[c]), :],
                load_sems.at[c % 2])

        def cast(c):
            send_x[c % S, pl.ds(0, SIZES[c]), :] = (
                in_vmem[c % 2, pl.ds(0, SIZES[c]), :].astype(jnp.bfloat16))

        def rdma_x(c):
            return pltpu.make_async_remote_copy(
                src_ref=send_x.at[c % S, pl.ds(0, SIZES[c]), :],
                dst_ref=recv_x.at[c % S, pl.ds(0, SIZES[c]), :],
                send_sem=sx_send.at[c % S], recv_sem=sx_recv.at[c % S],
                device_id=x_peer, device_id_type=pl.DeviceIdType.MESH)

        def rdma_y(c):
            return pltpu.make_async_remote_copy(
                src_ref=sum_buf.at[c % S, pl.ds(0, SIZES[c]), :],
                dst_ref=out_hbm.at[pl.ds(row0 + OFFS[c], SIZES[c]), :],
                send_sem=sy_send.at[c % S], recv_sem=sy_recv.at[c % S],
                device_id=y_peer, device_id_type=pl.DeviceIdType.MESH)

        def store(c):
            return pltpu.make_async_copy(
                sum_buf.at[c % S, pl.ds(0, SIZES[c]), :],
                out_hbm.at[pl.ds(row0 + OFFS[c], SIZES[c]), :],
                store_sems.at[c % S])

        def signal(sem, peer):
            pl.semaphore_signal(sem, inc=1, device_id=peer,
                                device_id_type=pl.DeviceIdType.MESH)

        load(0).start()
        load(0).wait()
        cast(0)
        load(1).start()
        rdma_x(0).start()
        load(1).wait()
        cast(1)
        load(2).start()

        for c in range(N):
            if c + 1 < N:
                if c + 1 >= S:
                    pl.semaphore_wait(credit_x, 1)
                rdma_x(c + 1).start()

            rdma_x(c).wait_recv()

            if c >= S:
                rdma_y(c - S).wait_send()
                store(c - S).wait()

            sum_buf[c % S, pl.ds(0, SIZES[c]), :] = (
                send_x[c % S, pl.ds(0, SIZES[c]), :]
                + recv_x[c % S, pl.ds(0, SIZES[c]), :])
            if c <= N - 1 - S:
                signal(credit_x, x_peer)

            if c >= S:
                pl.semaphore_wait(credit_y, 1)
            store(c).start()
            rdma_y(c).start()

            if c >= 1:
                rdma_y(c - 1).wait_recv()
                if c - 1 <= N - 1 - S:
                    signal(credit_y, y_peer)

            if c + 2 < N:
                load(c + 2).wait()
                if c + 2 - S >= 0:
                    rdma_x(c + 2 - S).wait_send()
                cast(c + 2)
                if c + 3 < N:
                    load(c + 3).start()

        rdma_y(N - 1).wait_recv()
        for k in range(max(0, N - S), N):
            rdma_x(k).wait_send()
            rdma_y(k).wait_send()
            store(k).wait()

    return pl.pallas_call(
        body,
        out_shape=jax.ShapeDtypeStruct((m, n), jnp.bfloat16),
        in_specs=[pl.BlockSpec(memory_space=pltpu.MemorySpace.HBM)],
        out_specs=pl.BlockSpec(memory_space=pltpu.MemorySpace.HBM),
        scratch_shapes=[
            pltpu.VMEM((2, MAXC, n), jnp.float32),
            pltpu.VMEM((S, MAXC, n), jnp.bfloat16),
            pltpu.VMEM((S, MAXC, n), jnp.bfloat16),
            pltpu.VMEM((S, MAXC, n), jnp.bfloat16),
            pltpu.SemaphoreType.DMA((2,)),
            pltpu.SemaphoreType.DMA((S,)),
            pltpu.SemaphoreType.DMA((S,)),
            pltpu.SemaphoreType.DMA((S,)),
            pltpu.SemaphoreType.DMA((S,)),
            pltpu.SemaphoreType.DMA((S,)),
            pltpu.SemaphoreType.REGULAR,
            pltpu.SemaphoreType.REGULAR,
        ],
    )(x)


# device time: 217801 ns/iter; 1.0774x vs baseline; 1.0774x over previous
import jax
import jax.numpy as jnp
from jax import lax
from jax.experimental import pallas as pl
from jax.experimental.pallas import tpu as pltpu

SIZES = [128, 128] + [256] * 30 + [128, 128]
OFFS = [sum(SIZES[:i]) for i in range(len(SIZES))]
N = len(SIZES)
MAXC = max(SIZES)
S = 4


def kernel(x):
    m, n = x.shape
    half = m // 2
    assert sum(SIZES) == half

    def body(x_hbm, out_hbm, in_vmem, send_x, recv_x, sum_buf,
             load_sems, store_sems, sx_send, sx_recv, sy_send, sy_recv,
             credit_x, credit_y):
        my_x = lax.axis_index("x")
        my_y = lax.axis_index("y")
        row0 = my_y * half
        x_peer = (1 - my_x, my_y)
        y_peer = (my_x, 1 - my_y)

        def load(c):
            return pltpu.make_async_copy(
                x_hbm.at[pl.ds(row0 + OFFS[c], SIZES[c]), :],
                in_vmem.at[c % 2, pl.ds(0, SIZES[c]), :],
                load_sems.at[c % 2])

        def cast(c):
            send_x[c % S, pl.ds(0, SIZES[c]), :] = (
                in_vmem[c % 2, pl.ds(0, SIZES[c]), :].astype(jnp.bfloat16))

        def rdma_x(c):
            return pltpu.make_async_remote_copy(
                src_ref=send_x.at[c % S, pl.ds(0, SIZES[c]), :],
                dst_ref=recv_x.at[c % S, pl.ds(0, SIZES[c]), :],
                send_sem=sx_send.at[c % S], recv_sem=sx_recv.at[c % S],
                device_id=x_peer, device_id_type=pl.DeviceIdType.MESH)

        def rdma_y(c):
            return pltpu.make_async_remote_copy(
                src_ref=sum_buf.at[c % S, pl.ds(0, SIZES[c]), :],
                dst_ref=out_hbm.at[pl.ds(row0 + OFFS[c], SIZES[c]), :],
                send_sem=sy_send.at[c % S], recv_sem=sy_recv.at[c % S],
                device_id=y_peer, device_id_type=pl.DeviceIdType.MESH)

        def store(c):
            return pltpu.make_async_copy(
                sum_buf.at[c % S, pl.ds(0, SIZES[c]), :],
                out_hbm.at[pl.ds(row0 + OFFS[c], SIZES[c]), :],
                store_sems.at[c % S])

        def signal(sem, peer):
            pl.semaphore_signal(sem, inc=1, device_id=peer,
                                device_id_type=pl.DeviceIdType.MESH)

        load(0).start()
        load(0).wait()
        cast(0)
        load(1).start()
        rdma_x(0).start()
        load(1).wait()
        cast(1)
        load(2).start()

        for c in range(N):
            if c + 1 < N:
                if c + 1 >= S:
                    pl.semaphore_wait(credit_x, 1)
                rdma_x(c + 1).start()

            rdma_x(c).wait_recv()

            if c >= S:
                rdma_y(c - S).wait_send()
                store(c - S).wait()

            sum_buf[c % S, pl.ds(0, SIZES[c]), :] = (
                send_x[c % S, pl.ds(0, SIZES[c]), :]
                + recv_x[c % S, pl.ds(0, SIZES[c]), :])
            if c <= N - 1 - S:
                signal(credit_x, x_peer)

            if c >= S:
                pl.semaphore_wait(credit_y, 1)
            store(c).start()
            rdma_y(c).start()

            if c >= 1:
                rdma_y(c - 1).wait_recv()
                if c - 1 <= N - 1 - S:
                    signal(credit_y, y_peer)

            if c + 2 < N:
                load(c + 2).wait()
                if c + 2 - S >= 0:
                    rdma_x(c + 2 - S).wait_send()
                cast(c + 2)
                if c + 3 < N:
                    load(c + 3).start()

        rdma_y(N - 1).wait_recv()
        for k in range(max(0, N - S), N):
            rdma_x(k).wait_send()
            rdma_y(k).wait_send()
            store(k).wait()

    return pl.pallas_call(
        body,
        out_shape=jax.ShapeDtypeStruct((m, n), jnp.bfloat16),
        in_specs=[pl.BlockSpec(memory_space=pltpu.MemorySpace.HBM)],
        out_specs=pl.BlockSpec(memory_space=pltpu.MemorySpace.HBM),
        scratch_shapes=[
            pltpu.VMEM((2, MAXC, n), jnp.float32),
            pltpu.VMEM((S, MAXC, n), jnp.bfloat16),
            pltpu.VMEM((S, MAXC, n), jnp.bfloat16),
            pltpu.VMEM((S, MAXC, n), jnp.bfloat16),
            pltpu.SemaphoreType.DMA((2,)),
            pltpu.SemaphoreType.DMA((S,)),
            pltpu.SemaphoreType.DMA((S,)),
            pltpu.SemaphoreType.DMA((S,)),
            pltpu.SemaphoreType.DMA((S,)),
            pltpu.SemaphoreType.DMA((S,)),
            pltpu.SemaphoreType.REGULAR,
            pltpu.SemaphoreType.REGULAR,
        ],
    )(x)
